# baseline (speedup 1.0000x reference)
import functools

import jax
import jax.numpy as jnp
from jax import lax
from jax.experimental import pallas as pl
from jax.experimental.pallas import tpu as pltpu
from jax.experimental.pallas import tpu_sc as plsc

_N = 10000
_E = 320000
_NC = 2
_NS = 16
_NW = _NC * _NS
_EPW = _E // _NW
_CHUNK = 80
_NCHUNK = _EPW // _CHUNK
_NBUF = 5
_NWB = 10
_WB = _N // _NWB
_ROW_BLK = 1000


def _make_segsum(D, chunk):
  assert D % 16 == 0
  nchunk = _EPW // chunk
  assert nchunk % _NBUF == 0
  mesh = plsc.VectorSubcoreMesh(core_axis_name="c", subcore_axis_name="s")

  @functools.partial(
      pl.kernel,
      out_type=jax.ShapeDtypeStruct((_NC, _N, D), jnp.float32),
      mesh=mesh,
      scratch_types=[
          pltpu.VMEM((nchunk, chunk), jnp.int32),
          pltpu.VMEM((nchunk, chunk), jnp.int32),
          [pltpu.VMEM((chunk, D), jnp.float32)] * _NBUF,
          pltpu.VMEM_SHARED((_N, D), jnp.float32),
          [pltpu.SemaphoreType.DMA] * _NBUF,
          [pltpu.SemaphoreType.DMA] * _NBUF,
      ],
      compiler_params=pltpu.CompilerParams(use_tc_tiling_on_sc=False),
  )
  def segsum(table_hbm, src_hbm, dst_hbm, out_hbm, src_v, dst_v, bufs,
             acc, gsems, ssems):
    buf = bufs[0]
    c = lax.axis_index("c")
    s = lax.axis_index("s")
    wid = c * _NS + s

    def zero_row(i, _):
      for k in range(D // 16):
        buf[i, pl.ds(k * 16, 16)] = jnp.zeros((16,), jnp.float32)
      return 0

    lax.fori_loop(0, chunk, zero_row, 0)

    row0 = s * _WB
    nfull = _WB // chunk
    rem = _WB - nfull * chunk

    @pl.when(s < _NWB)
    def _zero():
      for j in range(nfull):
        pltpu.sync_copy(buf, acc.at[pl.ds(row0 + j * chunk, chunk)])
      if rem:
        pltpu.sync_copy(buf.at[pl.ds(0, rem)],
                        acc.at[pl.ds(row0 + nfull * chunk, rem)])

    plsc.subcore_barrier()

    pltpu.sync_copy(src_hbm.at[wid], src_v)
    pltpu.sync_copy(dst_hbm.at[wid], dst_v)

    for k in range(_NBUF - 1):
      pltpu.async_copy(table_hbm.at[src_v.at[k]], bufs[k], gsems[k])

    def body(i, _):
      for k in range(_NBUF):
        j = i * _NBUF + k
        pltpu.make_async_copy(table_hbm.at[src_v.at[j]], bufs[k],
                              gsems[k]).wait()
        pltpu.async_copy(bufs[k], acc.at[dst_v.at[j]], ssems[k], add=True)
        m = (k + _NBUF - 1) % _NBUF

        @pl.when(j <= nchunk - _NBUF)
        def _issue():
          @pl.when(j >= 1)
          def _drain():
            pltpu.make_async_copy(bufs[m], acc.at[dst_v.at[0]],
                                  ssems[m]).wait()

          pltpu.async_copy(table_hbm.at[src_v.at[j + _NBUF - 1]], bufs[m],
                           gsems[m])

      return 0

    lax.fori_loop(0, nchunk // _NBUF, body, 0)
    for k in range(_NBUF):
      pltpu.make_async_copy(bufs[k], acc.at[dst_v.at[0]], ssems[k]).wait()
    plsc.subcore_barrier()

    @pl.when(s < _NWB)
    def _writeback():
      pltpu.sync_copy(acc.at[pl.ds(row0, _WB)],
                      out_hbm.at[c].at[pl.ds(row0, _WB)])

  return segsum


_CHUNK_128 = 40
_segsum_128 = _make_segsum(128, _CHUNK_128)
_segsum_64 = _make_segsum(64, _CHUNK)
_segsum_16 = _make_segsum(16, _CHUNK)


def _make_degsum():
  D = 16
  mesh = plsc.VectorSubcoreMesh(core_axis_name="c", subcore_axis_name="s")

  @functools.partial(
      pl.kernel,
      out_type=jax.ShapeDtypeStruct((_NC, _N, D), jnp.float32),
      mesh=mesh,
      scratch_types=[
          pltpu.VMEM((_NCHUNK, _CHUNK), jnp.int32),
          pltpu.VMEM((_CHUNK, D), jnp.float32),
          pltpu.VMEM((_CHUNK, D), jnp.float32),
          pltpu.VMEM_SHARED((_N, D), jnp.float32),
      ],
      compiler_params=pltpu.CompilerParams(use_tc_tiling_on_sc=False),
  )
  def degsum(dst_hbm, out_hbm, dst_v, zbuf, obuf, acc):
    c = lax.axis_index("c")
    s = lax.axis_index("s")
    wid = c * _NS + s

    def fill_row(i, _):
      zbuf[i, pl.ds(0, 16)] = jnp.zeros((16,), jnp.float32)
      obuf[i, pl.ds(0, 16)] = jnp.ones((16,), jnp.float32)
      return 0

    lax.fori_loop(0, _CHUNK, fill_row, 0)

    row0 = s * _WB
    nfull = _WB // _CHUNK
    rem = _WB - nfull * _CHUNK

    @pl.when(s < _NWB)
    def _zero():
      for j in range(nfull):
        pltpu.sync_copy(zbuf, acc.at[pl.ds(row0 + j * _CHUNK, _CHUNK)])
      if rem:
        pltpu.sync_copy(zbuf.at[pl.ds(0, rem)],
                        acc.at[pl.ds(row0 + nfull * _CHUNK, rem)])

    plsc.subcore_barrier()
    pltpu.sync_copy(dst_hbm.at[wid], dst_v)

    def body(j, _):
      pltpu.sync_copy(obuf, acc.at[dst_v.at[j]], add=True)
      return 0

    lax.fori_loop(0, _NCHUNK, body, 0)
    plsc.subcore_barrier()

    @pl.when(s < _NWB)
    def _writeback():
      pltpu.sync_copy(acc.at[pl.ds(row0, _WB)],
                      out_hbm.at[c].at[pl.ds(row0, _WB)])

  return degsum


_degsum = _make_degsum()


_GRID = _N // _ROW_BLK


def _tc_mm_body(x_ref, w1_ref, xw_ref):
  xw_ref[...] = jnp.dot(x_ref[...], w1_ref[...],
                        preferred_element_type=jnp.float32)


def _tc1_body(degp_ref, xw_ref, dis_ref, t1_ref):
  deg = degp_ref[0][:, 0:1] + degp_ref[1][:, 0:1] + 1.0
  dis = lax.rsqrt(deg)
  dis_ref[...] = dis
  t1_ref[...] = dis * xw_ref[...]


def _tc_mid_body(accp_ref, t_ref, dis_ref, w_ref, b_ref, out_ref):
  dis = dis_ref[...]
  ssum = accp_ref[0] + accp_ref[1] + t_ref[...]
  h = jnp.tanh(dis * ssum + b_ref[...])
  out_ref[...] = dis * jnp.dot(h, w_ref[...],
                               preferred_element_type=jnp.float32)


def _tc_out_body(accp_ref, t_ref, dis_ref, b_ref, out_ref):
  dis = dis_ref[...]
  full = dis * (accp_ref[0] + accp_ref[1] + t_ref[...]) + b_ref[...]
  out_ref[...] = full[:, 0:1]


def _row_spec(width):
  return pl.BlockSpec((_ROW_BLK, width), lambda i: (i, 0))


def _part_spec(width):
  return pl.BlockSpec((_NC, _ROW_BLK, width), lambda i: (0, i, 0))


def _full_spec(r, c):
  return pl.BlockSpec((r, c), lambda i: (0, 0))


def _tc_mm(x, w1):
  return pl.pallas_call(
      _tc_mm_body,
      grid=(_GRID,),
      in_specs=[_row_spec(128), _full_spec(128, 128)],
      out_specs=_row_spec(128),
      out_shape=jax.ShapeDtypeStruct((_N, 128), jnp.float32),
  )(x, w1)


def _tc1(degp, xw):
  return pl.pallas_call(
      _tc1_body,
      grid=(_GRID,),
      in_specs=[_part_spec(16), _row_spec(128)],
      out_specs=[_row_spec(1), _row_spec(128)],
      out_shape=[
          jax.ShapeDtypeStruct((_N, 1), jnp.float32),
          jax.ShapeDtypeStruct((_N, 128), jnp.float32),
      ],
  )(degp, xw)


def _tc_mid(accp, t, dis, w, b, d_in, d_out):
  return pl.pallas_call(
      _tc_mid_body,
      grid=(_GRID,),
      in_specs=[
          _part_spec(d_in), _row_spec(d_in), _row_spec(1),
          _full_spec(d_in, d_out), _full_spec(1, d_in),
      ],
      out_specs=_row_spec(d_out),
      out_shape=jax.ShapeDtypeStruct((_N, d_out), jnp.float32),
  )(accp, t, dis, w, b)


def _tc_out(accp, t, dis, b):
  return pl.pallas_call(
      _tc_out_body,
      grid=(_GRID,),
      in_specs=[
          _part_spec(16), _row_spec(16), _row_spec(1), _full_spec(1, 16),
      ],
      out_specs=_row_spec(1),
      out_shape=jax.ShapeDtypeStruct((_N, 1), jnp.float32),
  )(accp, t, dis, b)


@jax.jit
def kernel(x, edge_index, W1, b1, W2, b2, W3, b3):
  src2d = edge_index[0].reshape(_NW, _NCHUNK, _CHUNK)
  dst2d = edge_index[1].reshape(_NW, _NCHUNK, _CHUNK)
  src2d_40 = edge_index[0].reshape(_NW, _EPW // _CHUNK_128, _CHUNK_128)
  dst2d_40 = edge_index[1].reshape(_NW, _EPW // _CHUNK_128, _CHUNK_128)

  degp = _degsum(dst2d)
  xw1 = _tc_mm(x, W1)

  dis, t1 = _tc1(degp, xw1)
  acc1 = _segsum_128(t1, src2d_40, dst2d_40)
  t2 = _tc_mid(acc1, t1, dis, W2, b1.reshape(1, 128), 128, 64)
  w3p = jnp.pad(W3, ((0, 0), (0, 15)))
  t3 = _tc_mid(acc2 := _segsum_64(t2, src2d, dst2d), t2, dis, w3p,
               b2.reshape(1, 64), 64, 16)
  acc3 = _segsum_16(t3, src2d, dst2d)
  b3p = jnp.broadcast_to(b3.reshape(1, 1), (1, 16))
  return _tc_out(acc3, t3, dis, b3p)


# speedup vs baseline: 1.0015x; 1.0015x over previous
import functools

import jax
import jax.numpy as jnp
from jax import lax
from jax.experimental import pallas as pl
from jax.experimental.pallas import tpu as pltpu
from jax.experimental.pallas import tpu_sc as plsc

_N = 10000
_E = 320000
_NC = 2
_NS = 16
_NW = _NC * _NS
_EPW = _E // _NW
_CHUNK = 80
_NCHUNK = _EPW // _CHUNK
_NBUF = 5
_NWB = 10
_WB = _N // _NWB
_ROW_BLK = 1000


def _make_segsum(D, chunk):
  assert D % 16 == 0
  nchunk = _EPW // chunk
  assert nchunk % _NBUF == 0
  mesh = plsc.VectorSubcoreMesh(core_axis_name="c", subcore_axis_name="s")

  @functools.partial(
      pl.kernel,
      out_type=jax.ShapeDtypeStruct((_NC, _N, D), jnp.float32),
      mesh=mesh,
      scratch_types=[
          pltpu.VMEM((nchunk, chunk), jnp.int32),
          pltpu.VMEM((nchunk, chunk), jnp.int32),
          [pltpu.VMEM((chunk, D), jnp.float32)] * _NBUF,
          pltpu.VMEM_SHARED((_N, D), jnp.float32),
          [pltpu.SemaphoreType.DMA] * _NBUF,
          [pltpu.SemaphoreType.DMA] * _NBUF,
      ],
      compiler_params=pltpu.CompilerParams(use_tc_tiling_on_sc=False, skip_device_barrier=True),
  )
  def segsum(table_hbm, src_hbm, dst_hbm, out_hbm, src_v, dst_v, bufs,
             acc, gsems, ssems):
    buf = bufs[0]
    c = lax.axis_index("c")
    s = lax.axis_index("s")
    wid = c * _NS + s

    def zero_row(i, _):
      for k in range(D // 16):
        buf[i, pl.ds(k * 16, 16)] = jnp.zeros((16,), jnp.float32)
      return 0

    lax.fori_loop(0, chunk, zero_row, 0)

    row0 = s * _WB
    nfull = _WB // chunk
    rem = _WB - nfull * chunk

    @pl.when(s < _NWB)
    def _zero():
      for j in range(nfull):
        pltpu.sync_copy(buf, acc.at[pl.ds(row0 + j * chunk, chunk)])
      if rem:
        pltpu.sync_copy(buf.at[pl.ds(0, rem)],
                        acc.at[pl.ds(row0 + nfull * chunk, rem)])

    plsc.subcore_barrier()

    pltpu.sync_copy(src_hbm.at[wid], src_v)
    pltpu.sync_copy(dst_hbm.at[wid], dst_v)

    for k in range(_NBUF - 1):
      pltpu.async_copy(table_hbm.at[src_v.at[k]], bufs[k], gsems[k])

    def body(i, _):
      for k in range(_NBUF):
        j = i * _NBUF + k
        pltpu.make_async_copy(table_hbm.at[src_v.at[j]], bufs[k],
                              gsems[k]).wait()
        pltpu.async_copy(bufs[k], acc.at[dst_v.at[j]], ssems[k], add=True)
        m = (k + _NBUF - 1) % _NBUF

        @pl.when(j <= nchunk - _NBUF)
        def _issue():
          @pl.when(j >= 1)
          def _drain():
            pltpu.make_async_copy(bufs[m], acc.at[dst_v.at[0]],
                                  ssems[m]).wait()

          pltpu.async_copy(table_hbm.at[src_v.at[j + _NBUF - 1]], bufs[m],
                           gsems[m])

      return 0

    lax.fori_loop(0, nchunk // _NBUF, body, 0)
    for k in range(_NBUF):
      pltpu.make_async_copy(bufs[k], acc.at[dst_v.at[0]], ssems[k]).wait()
    plsc.subcore_barrier()

    @pl.when(s < _NWB)
    def _writeback():
      pltpu.sync_copy(acc.at[pl.ds(row0, _WB)],
                      out_hbm.at[c].at[pl.ds(row0, _WB)])

  return segsum


_CHUNK_128 = 40
_segsum_128 = _make_segsum(128, _CHUNK_128)
_segsum_64 = _make_segsum(64, _CHUNK)
_segsum_16 = _make_segsum(16, _CHUNK)


def _make_degsum():
  D = 16
  mesh = plsc.VectorSubcoreMesh(core_axis_name="c", subcore_axis_name="s")

  @functools.partial(
      pl.kernel,
      out_type=jax.ShapeDtypeStruct((_NC, _N, D), jnp.float32),
      mesh=mesh,
      scratch_types=[
          pltpu.VMEM((_NCHUNK, _CHUNK), jnp.int32),
          pltpu.VMEM((_CHUNK, D), jnp.float32),
          pltpu.VMEM((_CHUNK, D), jnp.float32),
          pltpu.VMEM_SHARED((_N, D), jnp.float32),
      ],
      compiler_params=pltpu.CompilerParams(use_tc_tiling_on_sc=False, skip_device_barrier=True),
  )
  def degsum(dst_hbm, out_hbm, dst_v, zbuf, obuf, acc):
    c = lax.axis_index("c")
    s = lax.axis_index("s")
    wid = c * _NS + s

    def fill_row(i, _):
      zbuf[i, pl.ds(0, 16)] = jnp.zeros((16,), jnp.float32)
      obuf[i, pl.ds(0, 16)] = jnp.ones((16,), jnp.float32)
      return 0

    lax.fori_loop(0, _CHUNK, fill_row, 0)

    row0 = s * _WB
    nfull = _WB // _CHUNK
    rem = _WB - nfull * _CHUNK

    @pl.when(s < _NWB)
    def _zero():
      for j in range(nfull):
        pltpu.sync_copy(zbuf, acc.at[pl.ds(row0 + j * _CHUNK, _CHUNK)])
      if rem:
        pltpu.sync_copy(zbuf.at[pl.ds(0, rem)],
                        acc.at[pl.ds(row0 + nfull * _CHUNK, rem)])

    plsc.subcore_barrier()
    pltpu.sync_copy(dst_hbm.at[wid], dst_v)

    def body(j, _):
      pltpu.sync_copy(obuf, acc.at[dst_v.at[j]], add=True)
      return 0

    lax.fori_loop(0, _NCHUNK, body, 0)
    plsc.subcore_barrier()

    @pl.when(s < _NWB)
    def _writeback():
      pltpu.sync_copy(acc.at[pl.ds(row0, _WB)],
                      out_hbm.at[c].at[pl.ds(row0, _WB)])

  return degsum


_degsum = _make_degsum()


_GRID = _N // _ROW_BLK


def _tc_mm_body(x_ref, w1_ref, xw_ref):
  xw_ref[...] = jnp.dot(x_ref[...], w1_ref[...],
                        preferred_element_type=jnp.float32)


def _tc1_body(degp_ref, xw_ref, dis_ref, t1_ref):
  deg = degp_ref[0][:, 0:1] + degp_ref[1][:, 0:1] + 1.0
  dis = lax.rsqrt(deg)
  dis_ref[...] = dis
  t1_ref[...] = dis * xw_ref[...]


def _tc_mid_body(accp_ref, t_ref, dis_ref, w_ref, b_ref, out_ref):
  dis = dis_ref[...]
  ssum = accp_ref[0] + accp_ref[1] + t_ref[...]
  h = jnp.tanh(dis * ssum + b_ref[...])
  out_ref[...] = dis * jnp.dot(h, w_ref[...],
                               preferred_element_type=jnp.float32)


def _tc_out_body(accp_ref, t_ref, dis_ref, b_ref, out_ref):
  dis = dis_ref[...]
  full = dis * (accp_ref[0] + accp_ref[1] + t_ref[...]) + b_ref[...]
  out_ref[...] = full[:, 0:1]


def _row_spec(width):
  return pl.BlockSpec((_ROW_BLK, width), lambda i: (i, 0))


def _part_spec(width):
  return pl.BlockSpec((_NC, _ROW_BLK, width), lambda i: (0, i, 0))


def _full_spec(r, c):
  return pl.BlockSpec((r, c), lambda i: (0, 0))


def _tc_mm(x, w1):
  return pl.pallas_call(
      _tc_mm_body,
      grid=(_GRID,),
      in_specs=[_row_spec(128), _full_spec(128, 128)],
      out_specs=_row_spec(128),
      out_shape=jax.ShapeDtypeStruct((_N, 128), jnp.float32),
  )(x, w1)


def _tc1(degp, xw):
  return pl.pallas_call(
      _tc1_body,
      grid=(_GRID,),
      in_specs=[_part_spec(16), _row_spec(128)],
      out_specs=[_row_spec(1), _row_spec(128)],
      out_shape=[
          jax.ShapeDtypeStruct((_N, 1), jnp.float32),
          jax.ShapeDtypeStruct((_N, 128), jnp.float32),
      ],
  )(degp, xw)


def _tc_mid(accp, t, dis, w, b, d_in, d_out):
  return pl.pallas_call(
      _tc_mid_body,
      grid=(_GRID,),
      in_specs=[
          _part_spec(d_in), _row_spec(d_in), _row_spec(1),
          _full_spec(d_in, d_out), _full_spec(1, d_in),
      ],
      out_specs=_row_spec(d_out),
      out_shape=jax.ShapeDtypeStruct((_N, d_out), jnp.float32),
  )(accp, t, dis, w, b)


def _tc_out(accp, t, dis, b):
  return pl.pallas_call(
      _tc_out_body,
      grid=(_GRID,),
      in_specs=[
          _part_spec(16), _row_spec(16), _row_spec(1), _full_spec(1, 16),
      ],
      out_specs=_row_spec(1),
      out_shape=jax.ShapeDtypeStruct((_N, 1), jnp.float32),
  )(accp, t, dis, b)


@jax.jit
def kernel(x, edge_index, W1, b1, W2, b2, W3, b3):
  src2d = edge_index[0].reshape(_NW, _NCHUNK, _CHUNK)
  dst2d = edge_index[1].reshape(_NW, _NCHUNK, _CHUNK)
  src2d_40 = edge_index[0].reshape(_NW, _EPW // _CHUNK_128, _CHUNK_128)
  dst2d_40 = edge_index[1].reshape(_NW, _EPW // _CHUNK_128, _CHUNK_128)

  degp = _degsum(dst2d)
  xw1 = _tc_mm(x, W1)

  dis, t1 = _tc1(degp, xw1)
  acc1 = _segsum_128(t1, src2d_40, dst2d_40)
  t2 = _tc_mid(acc1, t1, dis, W2, b1.reshape(1, 128), 128, 64)
  w3p = jnp.pad(W3, ((0, 0), (0, 15)))
  t3 = _tc_mid(acc2 := _segsum_64(t2, src2d, dst2d), t2, dis, w3p,
               b2.reshape(1, 64), 64, 16)
  acc3 = _segsum_16(t3, src2d, dst2d)
  b3p = jnp.broadcast_to(b3.reshape(1, 1), (1, 16))
  return _tc_out(acc3, t3, dis, b3p)


# speedup vs baseline: 1.0202x; 1.0186x over previous
import functools

import jax
import jax.numpy as jnp
from jax import lax
from jax.experimental import pallas as pl
from jax.experimental.pallas import tpu as pltpu
from jax.experimental.pallas import tpu_sc as plsc

_N = 10000
_E = 320000
_NC = 2
_NS = 16
_NW = _NC * _NS
_EPW = _E // _NW
_CHUNK = 80
_NCHUNK = _EPW // _CHUNK
_NBUF = 5
_NWB = 10
_WB = _N // _NWB
_ROW_BLK = 1000


def _make_segsum(D, chunk):
  assert D % 16 == 0
  nchunk = _EPW // chunk
  assert nchunk % _NBUF == 0
  mesh = plsc.VectorSubcoreMesh(core_axis_name="c", subcore_axis_name="s")

  @functools.partial(
      pl.kernel,
      out_type=jax.ShapeDtypeStruct((_NC, _N, D), jnp.float32),
      mesh=mesh,
      scratch_types=[
          pltpu.VMEM((nchunk, chunk), jnp.int32),
          pltpu.VMEM((nchunk, chunk), jnp.int32),
          [pltpu.VMEM((chunk, D), jnp.float32)] * _NBUF,
          pltpu.VMEM_SHARED((_N, D), jnp.float32),
          [pltpu.SemaphoreType.DMA] * _NBUF,
          [pltpu.SemaphoreType.DMA] * _NBUF,
      ],
      compiler_params=pltpu.CompilerParams(use_tc_tiling_on_sc=False),
  )
  def segsum(table_hbm, src_hbm, dst_hbm, out_hbm, src_v, dst_v, bufs,
             acc, gsems, ssems):
    buf = bufs[0]
    c = lax.axis_index("c")
    s = lax.axis_index("s")
    wid = c * _NS + s

    pltpu.async_copy(src_hbm.at[wid], src_v, gsems[0])
    pltpu.async_copy(dst_hbm.at[wid], dst_v, gsems[1])

    def zero_row(i, _):
      for k in range(D // 16):
        buf[i, pl.ds(k * 16, 16)] = jnp.zeros((16,), jnp.float32)
      return 0

    lax.fori_loop(0, chunk, zero_row, 0)

    row0 = s * _WB
    nfull = _WB // chunk
    rem = _WB - nfull * chunk

    @pl.when(s < _NWB)
    def _zero():
      for j in range(nfull):
        pltpu.sync_copy(buf, acc.at[pl.ds(row0 + j * chunk, chunk)])
      if rem:
        pltpu.sync_copy(buf.at[pl.ds(0, rem)],
                        acc.at[pl.ds(row0 + nfull * chunk, rem)])

    plsc.subcore_barrier()

    pltpu.make_async_copy(src_hbm.at[wid], src_v, gsems[0]).wait()
    pltpu.make_async_copy(dst_hbm.at[wid], dst_v, gsems[1]).wait()

    for k in range(_NBUF - 1):
      pltpu.async_copy(table_hbm.at[src_v.at[k]], bufs[k], gsems[k])

    def body(i, _):
      for k in range(_NBUF):
        j = i * _NBUF + k
        pltpu.make_async_copy(table_hbm.at[src_v.at[j]], bufs[k],
                              gsems[k]).wait()
        pltpu.async_copy(bufs[k], acc.at[dst_v.at[j]], ssems[k], add=True)
        m = (k + _NBUF - 1) % _NBUF

        @pl.when(j <= nchunk - _NBUF)
        def _issue():
          @pl.when(j >= 1)
          def _drain():
            pltpu.make_async_copy(bufs[m], acc.at[dst_v.at[0]],
                                  ssems[m]).wait()

          pltpu.async_copy(table_hbm.at[src_v.at[j + _NBUF - 1]], bufs[m],
                           gsems[m])

      return 0

    lax.fori_loop(0, nchunk // _NBUF, body, 0)
    for k in range(_NBUF):
      pltpu.make_async_copy(bufs[k], acc.at[dst_v.at[0]], ssems[k]).wait()
    plsc.subcore_barrier()

    @pl.when(s < _NWB)
    def _writeback():
      pltpu.sync_copy(acc.at[pl.ds(row0, _WB)],
                      out_hbm.at[c].at[pl.ds(row0, _WB)])

  return segsum


_CHUNK_128 = 40
_segsum_128 = _make_segsum(128, _CHUNK_128)
_segsum_64 = _make_segsum(64, _CHUNK)
_segsum_16 = _make_segsum(16, _CHUNK)


def _make_degsum():
  D = 16
  mesh = plsc.VectorSubcoreMesh(core_axis_name="c", subcore_axis_name="s")

  @functools.partial(
      pl.kernel,
      out_type=jax.ShapeDtypeStruct((_NC, _N, D), jnp.float32),
      mesh=mesh,
      scratch_types=[
          pltpu.VMEM((_NCHUNK, _CHUNK), jnp.int32),
          pltpu.VMEM((_CHUNK, D), jnp.float32),
          pltpu.VMEM((_CHUNK, D), jnp.float32),
          pltpu.VMEM_SHARED((_N, D), jnp.float32),
      ],
      compiler_params=pltpu.CompilerParams(use_tc_tiling_on_sc=False),
  )
  def degsum(dst_hbm, out_hbm, dst_v, zbuf, obuf, acc):
    c = lax.axis_index("c")
    s = lax.axis_index("s")
    wid = c * _NS + s

    def fill_row(i, _):
      zbuf[i, pl.ds(0, 16)] = jnp.zeros((16,), jnp.float32)
      obuf[i, pl.ds(0, 16)] = jnp.ones((16,), jnp.float32)
      return 0

    lax.fori_loop(0, _CHUNK, fill_row, 0)

    row0 = s * _WB
    nfull = _WB // _CHUNK
    rem = _WB - nfull * _CHUNK

    @pl.when(s < _NWB)
    def _zero():
      for j in range(nfull):
        pltpu.sync_copy(zbuf, acc.at[pl.ds(row0 + j * _CHUNK, _CHUNK)])
      if rem:
        pltpu.sync_copy(zbuf.at[pl.ds(0, rem)],
                        acc.at[pl.ds(row0 + nfull * _CHUNK, rem)])

    plsc.subcore_barrier()
    pltpu.sync_copy(dst_hbm.at[wid], dst_v)

    def body(j, _):
      pltpu.sync_copy(obuf, acc.at[dst_v.at[j]], add=True)
      return 0

    lax.fori_loop(0, _NCHUNK, body, 0)
    plsc.subcore_barrier()

    @pl.when(s < _NWB)
    def _writeback():
      pltpu.sync_copy(acc.at[pl.ds(row0, _WB)],
                      out_hbm.at[c].at[pl.ds(row0, _WB)])

  return degsum


_degsum = _make_degsum()


_GRID = _N // _ROW_BLK


def _tc_mm_body(x_ref, w1_ref, xw_ref):
  xw_ref[...] = jnp.dot(x_ref[...], w1_ref[...],
                        preferred_element_type=jnp.float32)


def _tc1_body(degp_ref, xw_ref, dis_ref, t1_ref):
  deg = degp_ref[0][:, 0:1] + degp_ref[1][:, 0:1] + 1.0
  dis = lax.rsqrt(deg)
  dis_ref[...] = dis
  t1_ref[...] = dis * xw_ref[...]


def _tc_mid_body(accp_ref, t_ref, dis_ref, w_ref, b_ref, out_ref):
  dis = dis_ref[...]
  ssum = accp_ref[0] + accp_ref[1] + t_ref[...]
  h = jnp.tanh(dis * ssum + b_ref[...])
  out_ref[...] = dis * jnp.dot(h, w_ref[...],
                               preferred_element_type=jnp.float32)


def _tc_out_body(accp_ref, t_ref, dis_ref, b_ref, out_ref):
  dis = dis_ref[...]
  full = dis * (accp_ref[0] + accp_ref[1] + t_ref[...]) + b_ref[...]
  out_ref[...] = full[:, 0:1]


def _row_spec(width):
  return pl.BlockSpec((_ROW_BLK, width), lambda i: (i, 0))


def _part_spec(width):
  return pl.BlockSpec((_NC, _ROW_BLK, width), lambda i: (0, i, 0))


def _full_spec(r, c):
  return pl.BlockSpec((r, c), lambda i: (0, 0))


def _tc_mm(x, w1):
  return pl.pallas_call(
      _tc_mm_body,
      grid=(_GRID,),
      in_specs=[_row_spec(128), _full_spec(128, 128)],
      out_specs=_row_spec(128),
      out_shape=jax.ShapeDtypeStruct((_N, 128), jnp.float32),
  )(x, w1)


def _tc1(degp, xw):
  return pl.pallas_call(
      _tc1_body,
      grid=(_GRID,),
      in_specs=[_part_spec(16), _row_spec(128)],
      out_specs=[_row_spec(1), _row_spec(128)],
      out_shape=[
          jax.ShapeDtypeStruct((_N, 1), jnp.float32),
          jax.ShapeDtypeStruct((_N, 128), jnp.float32),
      ],
  )(degp, xw)


def _tc_mid(accp, t, dis, w, b, d_in, d_out):
  return pl.pallas_call(
      _tc_mid_body,
      grid=(_GRID,),
      in_specs=[
          _part_spec(d_in), _row_spec(d_in), _row_spec(1),
          _full_spec(d_in, d_out), _full_spec(1, d_in),
      ],
      out_specs=_row_spec(d_out),
      out_shape=jax.ShapeDtypeStruct((_N, d_out), jnp.float32),
  )(accp, t, dis, w, b)


def _tc_out(accp, t, dis, b):
  return pl.pallas_call(
      _tc_out_body,
      grid=(_GRID,),
      in_specs=[
          _part_spec(16), _row_spec(16), _row_spec(1), _full_spec(1, 16),
      ],
      out_specs=_row_spec(1),
      out_shape=jax.ShapeDtypeStruct((_N, 1), jnp.float32),
  )(accp, t, dis, b)


@jax.jit
def kernel(x, edge_index, W1, b1, W2, b2, W3, b3):
  src2d = edge_index[0].reshape(_NW, _NCHUNK, _CHUNK)
  dst2d = edge_index[1].reshape(_NW, _NCHUNK, _CHUNK)
  src2d_40 = edge_index[0].reshape(_NW, _EPW // _CHUNK_128, _CHUNK_128)
  dst2d_40 = edge_index[1].reshape(_NW, _EPW // _CHUNK_128, _CHUNK_128)

  degp = _degsum(dst2d)
  xw1 = _tc_mm(x, W1)

  dis, t1 = _tc1(degp, xw1)
  acc1 = _segsum_128(t1, src2d_40, dst2d_40)
  t2 = _tc_mid(acc1, t1, dis, W2, b1.reshape(1, 128), 128, 64)
  w3p = jnp.pad(W3, ((0, 0), (0, 15)))
  t3 = _tc_mid(acc2 := _segsum_64(t2, src2d, dst2d), t2, dis, w3p,
               b2.reshape(1, 64), 64, 16)
  acc3 = _segsum_16(t3, src2d, dst2d)
  b3p = jnp.broadcast_to(b3.reshape(1, 1), (1, 16))
  return _tc_out(acc3, t3, dis, b3p)


# speedup vs baseline: 1.0359x; 1.0154x over previous
import functools

import jax
import jax.numpy as jnp
from jax import lax
from jax.experimental import pallas as pl
from jax.experimental.pallas import tpu as pltpu
from jax.experimental.pallas import tpu_sc as plsc

_N = 10000
_E = 320000
_NC = 2
_NS = 16
_NW = _NC * _NS
_EPW = _E // _NW
_CHUNK = 80
_NCHUNK = _EPW // _CHUNK
_NBUF = 5
_NWB = 10
_WB = _N // _NWB
_ROW_BLK = 1000


def _make_segsum(D, chunk):
  assert D % 16 == 0
  nchunk = _EPW // chunk
  assert nchunk % _NBUF == 0
  mesh = plsc.VectorSubcoreMesh(core_axis_name="c", subcore_axis_name="s")

  @functools.partial(
      pl.kernel,
      out_type=jax.ShapeDtypeStruct((_NC, _N, D), jnp.float32),
      mesh=mesh,
      scratch_types=[
          pltpu.VMEM((nchunk, chunk), jnp.int32),
          pltpu.VMEM((nchunk, chunk), jnp.int32),
          [pltpu.VMEM((chunk, D), jnp.float32)] * _NBUF,
          pltpu.VMEM_SHARED((_N, D), jnp.float32),
          [pltpu.SemaphoreType.DMA] * _NBUF,
          [pltpu.SemaphoreType.DMA] * _NBUF,
      ],
      compiler_params=pltpu.CompilerParams(use_tc_tiling_on_sc=False),
  )
  def segsum(table_hbm, src_hbm, dst_hbm, out_hbm, src_v, dst_v, bufs,
             acc, gsems, ssems):
    buf = bufs[0]
    c = lax.axis_index("c")
    s = lax.axis_index("s")
    wid = c * _NS + s

    pltpu.async_copy(src_hbm.at[wid], src_v, gsems[0])
    pltpu.async_copy(dst_hbm.at[wid], dst_v, gsems[1])

    def zero_row(i, _):
      for k in range(D // 16):
        buf[i, pl.ds(k * 16, 16)] = jnp.zeros((16,), jnp.float32)
      return 0

    lax.fori_loop(0, chunk, zero_row, 0)

    row0 = jnp.where(s < 14, s * 624, 8736 + (s - 14) * 632)

    def _zero_slice(nrows):
      nfull = nrows // chunk
      rem = nrows - nfull * chunk
      def go():
        for j in range(nfull):
          pltpu.sync_copy(buf, acc.at[pl.ds(row0 + j * chunk, chunk)])
        if rem:
          pltpu.sync_copy(buf.at[pl.ds(0, rem)],
                          acc.at[pl.ds(row0 + nfull * chunk, rem)])
      return go

    pl.when(s < 14)(_zero_slice(624))
    pl.when(s >= 14)(_zero_slice(632))

    plsc.subcore_barrier()

    pltpu.make_async_copy(src_hbm.at[wid], src_v, gsems[0]).wait()
    pltpu.make_async_copy(dst_hbm.at[wid], dst_v, gsems[1]).wait()

    for k in range(_NBUF - 1):
      pltpu.async_copy(table_hbm.at[src_v.at[k]], bufs[k], gsems[k])

    def body(i, _):
      for k in range(_NBUF):
        j = i * _NBUF + k
        pltpu.make_async_copy(table_hbm.at[src_v.at[j]], bufs[k],
                              gsems[k]).wait()
        pltpu.async_copy(bufs[k], acc.at[dst_v.at[j]], ssems[k], add=True)
        m = (k + _NBUF - 1) % _NBUF

        @pl.when(j <= nchunk - _NBUF)
        def _issue():
          @pl.when(j >= 1)
          def _drain():
            pltpu.make_async_copy(bufs[m], acc.at[dst_v.at[0]],
                                  ssems[m]).wait()

          pltpu.async_copy(table_hbm.at[src_v.at[j + _NBUF - 1]], bufs[m],
                           gsems[m])

      return 0

    lax.fori_loop(0, nchunk // _NBUF, body, 0)
    for k in range(_NBUF):
      pltpu.make_async_copy(bufs[k], acc.at[dst_v.at[0]], ssems[k]).wait()
    plsc.subcore_barrier()

    @pl.when(s < 14)
    def _writeback14():
      pltpu.sync_copy(acc.at[pl.ds(row0, 624)],
                      out_hbm.at[c].at[pl.ds(row0, 624)])

    @pl.when(s >= 14)
    def _writeback2():
      pltpu.sync_copy(acc.at[pl.ds(row0, 632)],
                      out_hbm.at[c].at[pl.ds(row0, 632)])

  return segsum


_CHUNK_128 = 40
_segsum_128 = _make_segsum(128, _CHUNK_128)
_segsum_64 = _make_segsum(64, _CHUNK)
_segsum_16 = _make_segsum(16, _CHUNK)


def _make_degsum():
  D = 16
  mesh = plsc.VectorSubcoreMesh(core_axis_name="c", subcore_axis_name="s")

  @functools.partial(
      pl.kernel,
      out_type=jax.ShapeDtypeStruct((_NC, _N, D), jnp.float32),
      mesh=mesh,
      scratch_types=[
          pltpu.VMEM((_NCHUNK, _CHUNK), jnp.int32),
          pltpu.VMEM((_CHUNK, D), jnp.float32),
          pltpu.VMEM((_CHUNK, D), jnp.float32),
          pltpu.VMEM_SHARED((_N, D), jnp.float32),
          pltpu.SemaphoreType.DMA,
      ],
      compiler_params=pltpu.CompilerParams(use_tc_tiling_on_sc=False),
  )
  def degsum(dst_hbm, out_hbm, dst_v, zbuf, obuf, acc, isem):
    c = lax.axis_index("c")
    s = lax.axis_index("s")
    wid = c * _NS + s
    pltpu.async_copy(dst_hbm.at[wid], dst_v, isem)

    def fill_row(i, _):
      zbuf[i, pl.ds(0, 16)] = jnp.zeros((16,), jnp.float32)
      obuf[i, pl.ds(0, 16)] = jnp.ones((16,), jnp.float32)
      return 0

    lax.fori_loop(0, _CHUNK, fill_row, 0)

    row0 = jnp.where(s < 14, s * 624, 8736 + (s - 14) * 632)

    def _zero_slice(nrows):
      nfull = nrows // _CHUNK
      rem = nrows - nfull * _CHUNK
      def go():
        for j in range(nfull):
          pltpu.sync_copy(zbuf, acc.at[pl.ds(row0 + j * _CHUNK, _CHUNK)])
        if rem:
          pltpu.sync_copy(zbuf.at[pl.ds(0, rem)],
                          acc.at[pl.ds(row0 + nfull * _CHUNK, rem)])
      return go

    pl.when(s < 14)(_zero_slice(624))
    pl.when(s >= 14)(_zero_slice(632))

    plsc.subcore_barrier()
    pltpu.make_async_copy(dst_hbm.at[wid], dst_v, isem).wait()

    def body(j, _):
      pltpu.sync_copy(obuf, acc.at[dst_v.at[j]], add=True)
      return 0

    lax.fori_loop(0, _NCHUNK, body, 0)
    plsc.subcore_barrier()

    @pl.when(s < 14)
    def _writeback14():
      pltpu.sync_copy(acc.at[pl.ds(row0, 624)],
                      out_hbm.at[c].at[pl.ds(row0, 624)])

    @pl.when(s >= 14)
    def _writeback2():
      pltpu.sync_copy(acc.at[pl.ds(row0, 632)],
                      out_hbm.at[c].at[pl.ds(row0, 632)])

  return degsum


_degsum = _make_degsum()


_GRID = _N // _ROW_BLK


def _tc_mm_body(x_ref, w1_ref, xw_ref):
  xw_ref[...] = jnp.dot(x_ref[...], w1_ref[...],
                        preferred_element_type=jnp.float32)


def _tc1_body(degp_ref, xw_ref, dis_ref, t1_ref):
  deg = degp_ref[0][:, 0:1] + degp_ref[1][:, 0:1] + 1.0
  dis = lax.rsqrt(deg)
  dis_ref[...] = dis
  t1_ref[...] = dis * xw_ref[...]


def _tc_mid_body(accp_ref, t_ref, dis_ref, w_ref, b_ref, out_ref):
  dis = dis_ref[...]
  ssum = accp_ref[0] + accp_ref[1] + t_ref[...]
  h = jnp.tanh(dis * ssum + b_ref[...])
  out_ref[...] = dis * jnp.dot(h, w_ref[...],
                               preferred_element_type=jnp.float32)


def _tc_out_body(accp_ref, t_ref, dis_ref, b_ref, out_ref):
  dis = dis_ref[...]
  full = dis * (accp_ref[0] + accp_ref[1] + t_ref[...]) + b_ref[...]
  out_ref[...] = full[:, 0:1]


def _row_spec(width):
  return pl.BlockSpec((_ROW_BLK, width), lambda i: (i, 0))


def _part_spec(width):
  return pl.BlockSpec((_NC, _ROW_BLK, width), lambda i: (0, i, 0))


def _full_spec(r, c):
  return pl.BlockSpec((r, c), lambda i: (0, 0))


def _tc_mm(x, w1):
  return pl.pallas_call(
      _tc_mm_body,
      grid=(_GRID,),
      in_specs=[_row_spec(128), _full_spec(128, 128)],
      out_specs=_row_spec(128),
      out_shape=jax.ShapeDtypeStruct((_N, 128), jnp.float32),
  )(x, w1)


def _tc1(degp, xw):
  return pl.pallas_call(
      _tc1_body,
      grid=(_GRID,),
      in_specs=[_part_spec(16), _row_spec(128)],
      out_specs=[_row_spec(1), _row_spec(128)],
      out_shape=[
          jax.ShapeDtypeStruct((_N, 1), jnp.float32),
          jax.ShapeDtypeStruct((_N, 128), jnp.float32),
      ],
  )(degp, xw)


def _tc_mid(accp, t, dis, w, b, d_in, d_out):
  return pl.pallas_call(
      _tc_mid_body,
      grid=(_GRID,),
      in_specs=[
          _part_spec(d_in), _row_spec(d_in), _row_spec(1),
          _full_spec(d_in, d_out), _full_spec(1, d_in),
      ],
      out_specs=_row_spec(d_out),
      out_shape=jax.ShapeDtypeStruct((_N, d_out), jnp.float32),
  )(accp, t, dis, w, b)


def _tc_out(accp, t, dis, b):
  return pl.pallas_call(
      _tc_out_body,
      grid=(_GRID,),
      in_specs=[
          _part_spec(16), _row_spec(16), _row_spec(1), _full_spec(1, 16),
      ],
      out_specs=_row_spec(1),
      out_shape=jax.ShapeDtypeStruct((_N, 1), jnp.float32),
  )(accp, t, dis, b)


@jax.jit
def kernel(x, edge_index, W1, b1, W2, b2, W3, b3):
  src2d = edge_index[0].reshape(_NW, _NCHUNK, _CHUNK)
  dst2d = edge_index[1].reshape(_NW, _NCHUNK, _CHUNK)
  src2d_40 = edge_index[0].reshape(_NW, _EPW // _CHUNK_128, _CHUNK_128)
  dst2d_40 = edge_index[1].reshape(_NW, _EPW // _CHUNK_128, _CHUNK_128)

  degp = _degsum(dst2d)
  xw1 = _tc_mm(x, W1)

  dis, t1 = _tc1(degp, xw1)
  acc1 = _segsum_128(t1, src2d_40, dst2d_40)
  t2 = _tc_mid(acc1, t1, dis, W2, b1.reshape(1, 128), 128, 64)
  w3p = jnp.pad(W3, ((0, 0), (0, 15)))
  t3 = _tc_mid(acc2 := _segsum_64(t2, src2d, dst2d), t2, dis, w3p,
               b2.reshape(1, 64), 64, 16)
  acc3 = _segsum_16(t3, src2d, dst2d)
  b3p = jnp.broadcast_to(b3.reshape(1, 1), (1, 16))
  return _tc_out(acc3, t3, dis, b3p)
